# baseline (speedup 1.0000x reference)
import jax
import jax.numpy as jnp
from jax.experimental import pallas as pl


def _counts(ei, n):
    return jax.ops.segment_sum(
        jnp.ones((ei.shape[1],), jnp.float32), ei[1], num_segments=n)


def _agg(x_src, ei, n):
    return jax.ops.segment_sum(x_src[ei[0]], ei[1], num_segments=n)


def kernel(x_lncrna, x_mirna, x_protein, edge_index_lpi, edge_index_rev_lpi, edge_index_lmi, edge_index_rev_lmi, edge_index_mpi, edge_index_rev_mpi, edge_label_index, W_lnc_emb, b_lnc_emb, W_mir_emb, b_mir_emb, W_pro_emb, b_pro_emb, Wl, bl, Wr, W_c1, b_c1, W_c2, b_c2, W_c3, b_c3, W_h1, b_h1, W_h2, b_h2, W_h3, b_h3):
    N_L, N_M, N_P = x_lncrna.shape[0], x_mirna.shape[0], x_protein.shape[0]
    lpi, rlpi = edge_index_lpi, edge_index_rev_lpi
    lmi, rlmi = edge_index_lmi, edge_index_rev_lmi
    mpi, rmpi = edge_index_mpi, edge_index_rev_mpi

    r_lpi = 1.0 / jnp.clip(_counts(lpi, N_P), 1.0)
    r_rlpi = 1.0 / jnp.clip(_counts(rlpi, N_L), 1.0)
    r_lmi = 1.0 / jnp.clip(_counts(lmi, N_M), 1.0)
    r_rlmi = 1.0 / jnp.clip(_counts(rlmi, N_L), 1.0)
    r_mpi = 1.0 / jnp.clip(_counts(mpi, N_P), 1.0)
    r_rmpi = 1.0 / jnp.clip(_counts(rmpi, N_M), 1.0)

    hl = x_lncrna @ W_lnc_emb + b_lnc_emb
    hm = x_mirna @ W_mir_emb + b_mir_emb
    hp = x_protein @ W_pro_emb + b_pro_emb

    for li in range(2):
        ol = ((_agg(hp, rlpi, N_L) * r_rlpi[:, None]) @ Wl[li, 1]
              + (_agg(hm, rlmi, N_L) * r_rlmi[:, None]) @ Wl[li, 3]
              + hl @ (Wr[li, 1] + Wr[li, 3]) + bl[li, 1] + bl[li, 3])
        om = ((_agg(hl, lmi, N_M) * r_lmi[:, None]) @ Wl[li, 2]
              + (_agg(hp, rmpi, N_M) * r_rmpi[:, None]) @ Wl[li, 5]
              + hm @ (Wr[li, 2] + Wr[li, 5]) + bl[li, 2] + bl[li, 5])
        op = ((_agg(hl, lpi, N_P) * r_lpi[:, None]) @ Wl[li, 0]
              + (_agg(hm, mpi, N_P) * r_mpi[:, None]) @ Wl[li, 4]
              + hp @ (Wr[li, 0] + Wr[li, 4]) + bl[li, 0] + bl[li, 4])
        hl, hm, hp = jax.nn.relu(ol), jax.nn.relu(om), jax.nn.relu(op)

    hp3 = ((_agg(hl, lpi, N_P) * r_lpi[:, None]) @ Wl[2, 0]
           + (_agg(hm, mpi, N_P) * r_mpi[:, None]) @ Wl[2, 4]
           + hp @ (Wr[2, 0] + Wr[2, 4]) + bl[2, 0] + bl[2, 4])

    W_h1a, W_h1b = W_h1[:150], W_h1[150:]
    t = jax.nn.relu(x_lncrna @ W_c1 + b_c1)
    t = jax.nn.relu(t @ W_c2 + b_c2)
    a = t @ (W_c3 @ W_h1a) + (b_c3 @ W_h1a + b_h1)
    b = hp3 @ W_h1b

    ef = jax.nn.relu(a[edge_label_index[0]] + b[edge_label_index[1]])
    ef = jax.nn.relu(ef @ W_h2 + b_h2)
    return ef @ W_h3 + b_h3


# speedup vs baseline: 1.0569x; 1.0569x over previous
import jax
import jax.numpy as jnp
from jax import lax
from jax.experimental import pallas as pl
from jax.experimental.pallas import tpu as pltpu
from jax.experimental.pallas import tpu_sc as plsc

_B = 128
_W = 128
_NCH = 2
_NW = 32
_H = 150
_HPAD = 256
_NLOC = 10000
_NBSEG = 23


def _ceil(a, b):
    return -(-a // b)


def _row_alloc(n_loc):
    rpt = 8 * _ceil(n_loc + 1, 16 * 8)
    return rpt * 16, rpt


def _zero_segs(rpt):
    segs, off = [], 0
    while off < rpt:
        s = min(_B, rpt - off)
        segs.append((off, s))
        off += s
    return segs


def _pack_col(col, pad_val):
    e = col.shape[0]
    nbt = _ceil(e, _NW * _B)
    epad = nbt * _NW * _B
    col = jnp.concatenate(
        [col, jnp.full((epad - e,), pad_val, jnp.int32)]).reshape(_NW, nbt, _B)
    segs, o = [], 0
    while o < nbt:
        s = min(_NBSEG, nbt - o)
        segs.append((col[:, o:o + s], s))
        o += s
    return segs


def _segsum_launch(tasks, xs_by_type):
    rows_max = max(_row_alloc(t[5])[0] for t in tasks)
    nb_max = _NBSEG
    types_used = sorted({t[1] for t in tasks})

    in_args = [jnp.zeros((_B, _W), jnp.float32)]
    seen = {}
    for (_, _, _, ssegs, dsegs, _) in tasks:
        for segs in (ssegs, dsegs):
            if id(segs) not in seen:
                seen[id(segs)] = len(in_args)
                for (arr, _) in segs:
                    in_args.append(arr)
    type_pos = {}
    for t in types_used:
        for c in range(_NCH):
            type_pos[(t, c)] = len(in_args)
            in_args.append(xs_by_type[t][c])

    out_type = [jax.ShapeDtypeStruct((2, _row_alloc(t[5])[0], _W),
                                     jnp.float32) for t in tasks]
    n_in = len(in_args)

    def body(*refs):
        zhbm = refs[0]

        def seg_refs(segs):
            p = seen[id(segs)]
            return [(refs[p + i], segs[i][1]) for i in range(len(segs))]

        out_refs = refs[n_in:n_in + len(tasks)]
        sidx, didx, gbuf, zbuf, acc = refs[n_in + len(tasks):]

        cid = lax.axis_index("c")
        sid = lax.axis_index("s")
        wid = cid * 16 + sid

        pltpu.sync_copy(zhbm, zbuf)

        for ti, (key, src_t, ch, ssegs, dsegs, n_loc) in enumerate(tasks):
            table = refs[type_pos[(src_t, ch)]]
            out = out_refs[ti]
            _, rpt = _row_alloc(n_loc)
            row0 = sid * rpt
            for (o, s) in _zero_segs(rpt):
                pltpu.sync_copy(zbuf.at[pl.ds(0, s)],
                                acc.at[pl.ds(row0 + o, s)])
            plsc.subcore_barrier()

            for (sref, nb), (dref, nb2) in zip(seg_refs(ssegs),
                                               seg_refs(dsegs)):
                assert nb == nb2
                pltpu.sync_copy(sref.at[wid], sidx.at[pl.ds(0, nb)])
                pltpu.sync_copy(dref.at[wid], didx.at[pl.ds(0, nb)])

                @pl.loop(0, nb)
                def _(j):
                    pltpu.sync_copy(table.at[sidx.at[j]], gbuf)
                    pltpu.sync_copy(gbuf, acc.at[didx.at[j]], add=True)

            plsc.subcore_barrier()
            pltpu.sync_copy(acc.at[pl.ds(row0, rpt)],
                            out.at[cid, pl.ds(row0, rpt)])
            plsc.subcore_barrier()

    fn = pl.kernel(
        body,
        out_type=out_type,
        mesh=plsc.VectorSubcoreMesh(core_axis_name="c", subcore_axis_name="s"),
        compiler_params=pltpu.CompilerParams(use_tc_tiling_on_sc=False),
        scratch_types=[
            pltpu.VMEM((nb_max, _B), jnp.int32),
            pltpu.VMEM((nb_max, _B), jnp.int32),
            pltpu.VMEM((_B, _W), jnp.float32),
            pltpu.VMEM((_B, _W), jnp.float32),
            pltpu.VMEM_SHARED((rows_max, _W), jnp.float32),
        ],
    )
    outs = fn(*in_args)
    return {t[0]: o for t, o in zip(tasks, outs)}


def _b2(b, n):
    z = jnp.zeros((8, n), jnp.float32)
    return z.at[0, :b.shape[0]].set(b)


def _embed_tc(x, W, b2, bm=1000):
    n, k = x.shape

    def body(x_ref, w_ref, b_ref, a_ref, bb_ref):
        h = jnp.dot(x_ref[...], w_ref[...],
                    preferred_element_type=jnp.float32) + b_ref[0:1, :]
        a_ref[...] = h[:, :_W]
        bb_ref[...] = h[:, _W:]

    return pl.pallas_call(
        body,
        grid=(n // bm,),
        in_specs=[
            pl.BlockSpec((bm, k), lambda i: (i, 0)),
            pl.BlockSpec((k, _HPAD), lambda i: (0, 0)),
            pl.BlockSpec((8, _HPAD), lambda i: (0, 0)),
        ],
        out_specs=[pl.BlockSpec((bm, _W), lambda i: (i, 0))] * 2,
        out_shape=[jax.ShapeDtypeStruct((n, _W), jnp.float32)] * 2,
    )(x, W, b2)


def _lnc_tc(x, Wemb, bemb2, Wc1, bc12, Wc2, bc22, Wc3h, bc3h2, bm=1000):
    n, k = x.shape
    na = Wc3h.shape[1]

    def body(x_ref, we_ref, be_ref, w1_ref, b1_ref, w2_ref, b2_ref,
             w3_ref, b3_ref, a_ref, bb_ref, o_ref):
        xv = x_ref[...]
        h = jnp.dot(xv, we_ref[...],
                    preferred_element_type=jnp.float32) + be_ref[0:1, :]
        a_ref[...] = h[:, :_W]
        bb_ref[...] = h[:, _W:]
        t = jax.nn.relu(jnp.dot(xv, w1_ref[...],
                                preferred_element_type=jnp.float32)
                        + b1_ref[0:1, :])
        t = jax.nn.relu(jnp.dot(t, w2_ref[...],
                                preferred_element_type=jnp.float32)
                        + b2_ref[0:1, :])
        o_ref[...] = jnp.dot(t, w3_ref[...],
                             preferred_element_type=jnp.float32) + b3_ref[0:1, :]

    n1, n2 = Wc1.shape[1], Wc2.shape[1]
    return pl.pallas_call(
        body,
        grid=(n // bm,),
        in_specs=[
            pl.BlockSpec((bm, k), lambda i: (i, 0)),
            pl.BlockSpec((k, _HPAD), lambda i: (0, 0)),
            pl.BlockSpec((8, _HPAD), lambda i: (0, 0)),
            pl.BlockSpec((k, n1), lambda i: (0, 0)),
            pl.BlockSpec((8, n1), lambda i: (0, 0)),
            pl.BlockSpec((n1, n2), lambda i: (0, 0)),
            pl.BlockSpec((8, n2), lambda i: (0, 0)),
            pl.BlockSpec((n2, na), lambda i: (0, 0)),
            pl.BlockSpec((8, na), lambda i: (0, 0)),
        ],
        out_specs=[pl.BlockSpec((bm, _W), lambda i: (i, 0)),
                   pl.BlockSpec((bm, _W), lambda i: (i, 0)),
                   pl.BlockSpec((bm, na), lambda i: (i, 0))],
        out_shape=[jax.ShapeDtypeStruct((n, _W), jnp.float32),
                   jax.ShapeDtypeStruct((n, _W), jnp.float32),
                   jax.ShapeDtypeStruct((n, na), jnp.float32)],
    )(x, Wemb, bemb2, Wc1, bc12, Wc2, bc22, Wc3h, bc3h2)


def _head_tc(ag, bg, Wh2, bh22, Wh3, bh32, bm=1000):
    e, d = ag.shape
    n2, n3 = Wh2.shape[1], Wh3.shape[1]

    def body(a_ref, b_ref, w2_ref, b2_ref, w3_ref, b3_ref, o_ref):
        ef = jax.nn.relu(a_ref[...] + b_ref[...])
        ef = jax.nn.relu(jnp.dot(ef, w2_ref[...],
                                 preferred_element_type=jnp.float32)
                         + b2_ref[0:1, :])
        o_ref[...] = jnp.dot(ef, w3_ref[...],
                             preferred_element_type=jnp.float32) + b3_ref[0:1, :]

    return pl.pallas_call(
        body,
        grid=(e // bm,),
        in_specs=[
            pl.BlockSpec((bm, d), lambda i: (i, 0)),
            pl.BlockSpec((bm, d), lambda i: (i, 0)),
            pl.BlockSpec((d, n2), lambda i: (0, 0)),
            pl.BlockSpec((8, n2), lambda i: (0, 0)),
            pl.BlockSpec((n2, n3), lambda i: (0, 0)),
            pl.BlockSpec((8, n3), lambda i: (0, 0)),
        ],
        out_specs=pl.BlockSpec((bm, n3), lambda i: (i, 0)),
        out_shape=jax.ShapeDtypeStruct((e, n3), jnp.float32),
    )(ag, bg, Wh2, bh22, Wh3, bh32)


def _pad_h(h, ones_col):
    n = h.shape[0]
    pad = jnp.zeros((n, _HPAD - _H), h.dtype)
    if ones_col:
        pad = pad.at[:, 0].set(1.0)
    full = jnp.concatenate([h, pad], axis=1)
    return [full[:, :_W], full[:, _W:]]


class _EdgeIdx:
    def __init__(self, ei, n_dst):
        src, dst = ei[0], ei[1]
        self.n_dst = n_dst
        self.src_segs = _pack_col(src, 0)
        npass = _ceil(n_dst, _NLOC)
        self.quarters = []
        for q in range(npass):
            lo = q * _NLOC
            loc = jnp.where((dst >= lo) & (dst < lo + _NLOC),
                            dst - lo, _NLOC)
            self.quarters.append(_pack_col(loc, _NLOC))
        self.n_loc = _NLOC


def _tasks_for(ekey, eidx, src_t):
    ts = []
    for c in range(_NCH):
        for q, dsegs in enumerate(eidx.quarters):
            ts.append(((ekey, c, q), src_t, c, eidx.src_segs, dsegs,
                       eidx.n_loc))
    return ts


def _merge_agg(res, ekey, eidx):
    cols = []
    for c in range(_NCH):
        parts = []
        for q in range(len(eidx.quarters)):
            o = res[(ekey, c, q)]
            parts.append((o[0] + o[1])[:eidx.n_loc])
        cc = jnp.concatenate(parts, axis=0) if len(parts) > 1 else parts[0]
        cols.append(cc[:eidx.n_dst])
    return jnp.concatenate(cols, axis=1)


def kernel(x_lncrna, x_mirna, x_protein, edge_index_lpi, edge_index_rev_lpi, edge_index_lmi, edge_index_rev_lmi, edge_index_mpi, edge_index_rev_mpi, edge_label_index, W_lnc_emb, b_lnc_emb, W_mir_emb, b_mir_emb, W_pro_emb, b_pro_emb, Wl, bl, Wr, W_c1, b_c1, W_c2, b_c2, W_c3, b_c3, W_h1, b_h1, W_h2, b_h2, W_h3, b_h3):
    N_L, N_M, N_P = x_lncrna.shape[0], x_mirna.shape[0], x_protein.shape[0]
    n_of = {0: N_L, 1: N_M, 2: N_P}

    eis = {
        "lpi": (edge_index_lpi, 0, 2), "rlpi": (edge_index_rev_lpi, 2, 0),
        "lmi": (edge_index_lmi, 0, 1), "rlmi": (edge_index_rev_lmi, 1, 0),
        "mpi": (edge_index_mpi, 1, 2), "rmpi": (edge_index_rev_mpi, 2, 1),
    }
    idx = {k: _EdgeIdx(ei, n_of[dt]) for k, (ei, _, dt) in eis.items()}

    def wpad(W, b, ones_col):
        k = W.shape[0]
        Wp = jnp.zeros((k, _HPAD), jnp.float32).at[:, :_H].set(W)
        b2 = _b2(b, _HPAD)
        if ones_col:
            b2 = b2.at[0, _H].set(1.0)
        return Wp, b2

    W_h1a, W_h1b = W_h1[:_H], W_h1[_H:]
    NA = 152
    Wc3h = jnp.zeros((W_c2.shape[1], NA), jnp.float32).at[:, :_H].set(
        W_c3 @ W_h1a)
    bc3h2 = _b2(b_c3 @ W_h1a + b_h1, NA)

    Wle, ble2 = wpad(W_lnc_emb, b_lnc_emb, True)
    Wme, bme2 = wpad(W_mir_emb, b_mir_emb, True)
    Wpe, bpe2 = wpad(W_pro_emb, b_pro_emb, True)

    lA, lB, a = _lnc_tc(x_lncrna, Wle, ble2, W_c1, _b2(b_c1, 300),
                        W_c2, _b2(b_c2, 200), Wc3h, bc3h2)
    mA, mB = _embed_tc(x_mirna, Wme, bme2)
    pA, pB = _embed_tc(x_protein, Wpe, bpe2)
    tabs = {0: [lA, lB], 1: [mA, mB], 2: [pA, pB]}

    def xterm(tt, Wr_sum):
        A, Bc = tt
        Wb = jnp.zeros((_W, _H), jnp.float32).at[:_H - _W].set(
            Wr_sum[_W:_H])
        return A @ Wr_sum[:_W] + Bc @ Wb

    rcp = {}

    def consume(res, ekey, layer):
        full = _merge_agg(res, ekey, idx[ekey])
        if layer == 0:
            rcp[ekey] = 1.0 / jnp.clip(full[:, _H], 1.0)
        return full[:, :_H] * rcp[ekey][:, None]

    for li in range(3):
        if li < 2:
            tasks = []
            for k, (_, st, _) in eis.items():
                tasks += _tasks_for(k, idx[k], st)
            res = _segsum_launch(tasks, tabs)
            ol = (consume(res, "rlpi", li) @ Wl[li, 1]
                  + consume(res, "rlmi", li) @ Wl[li, 3]
                  + xterm(tabs[0], Wr[li, 1] + Wr[li, 3])
                  + bl[li, 1] + bl[li, 3])
            om = (consume(res, "lmi", li) @ Wl[li, 2]
                  + consume(res, "rmpi", li) @ Wl[li, 5]
                  + xterm(tabs[1], Wr[li, 2] + Wr[li, 5])
                  + bl[li, 2] + bl[li, 5])
            op = (consume(res, "lpi", li) @ Wl[li, 0]
                  + consume(res, "mpi", li) @ Wl[li, 4]
                  + xterm(tabs[2], Wr[li, 0] + Wr[li, 4])
                  + bl[li, 0] + bl[li, 4])
            tabs = {0: _pad_h(jax.nn.relu(ol), False),
                    1: _pad_h(jax.nn.relu(om), False),
                    2: _pad_h(jax.nn.relu(op), False)}
        else:
            tasks = (_tasks_for("lpi", idx["lpi"], 0)
                     + _tasks_for("mpi", idx["mpi"], 1))
            res = _segsum_launch(tasks, {0: tabs[0], 1: tabs[1]})
            hp3 = (consume(res, "lpi", li) @ Wl[2, 0]
                   + consume(res, "mpi", li) @ Wl[2, 4]
                   + xterm(tabs[2], Wr[2, 0] + Wr[2, 4])
                   + bl[2, 0] + bl[2, 4])

    b = hp3 @ W_h1b
    b = jnp.concatenate([b, jnp.zeros((N_P, NA - _H), jnp.float32)], axis=1)
    ag = a[edge_label_index[0]]
    bg = b[edge_label_index[1]]
    Wh2p = jnp.zeros((NA, 64), jnp.float32).at[:_H, :50].set(W_h2)
    Wh3p = jnp.zeros((64, 8), jnp.float32).at[:50, :3].set(W_h3)
    out = _head_tc(ag, bg, Wh2p, _b2(b_h2, 64), Wh3p, _b2(b_h3, 8))
    return out[:, :3]
